# Initial kernel scaffold; baseline (speedup 1.0000x reference)
import jax
import jax.numpy as jnp
from jax.experimental import pallas as pl


def kernel(x, edge_index, W1, b1, W2, b2):
    raise NotImplementedError("write your pallas kernel here")


# speedup vs baseline: 12.7218x; 12.7218x over previous
import functools

import jax
import jax.numpy as jnp
from jax import lax
from jax.experimental import pallas as pl
from jax.experimental.pallas import tpu as pltpu
from jax.experimental.pallas import tpu_sc as plsc

N = 10000
D = 128
E = 320000

NC = 2
NS = 16
NW = NC * NS
EDGES_PER_W = E // NW
CHUNK = 80
NCHUNK = EDGES_PER_W // CHUNK
N_PAD = 10240
ROWS_PER_S = N_PAD // NS
ZROWS = 128
ZCOPIES = ROWS_PER_S // ZROWS
HW = 16

def _deg_body(dst_hbm, out_hbm, idx_v, ones_v, zbuf_v, hist_sh):
    cid = lax.axis_index("c")
    sid = lax.axis_index("s")
    wid = sid * NC + cid

    @pl.loop(0, CHUNK)
    def _(r):
        ones_v[r, :] = jnp.ones((HW,), jnp.float32)

    @pl.loop(0, ZROWS)
    def _(r):
        zbuf_v[r, :] = jnp.zeros((HW,), jnp.float32)

    @pl.loop(0, ZCOPIES)
    def _(k):
        pltpu.sync_copy(
            zbuf_v, hist_sh.at[pl.ds(sid * ROWS_PER_S + k * ZROWS, ZROWS)])

    plsc.subcore_barrier()

    base = wid * EDGES_PER_W

    @pl.loop(0, NCHUNK)
    def _(j):
        off = pl.multiple_of(base + j * CHUNK, 8)
        pltpu.sync_copy(dst_hbm.at[pl.ds(off, CHUNK)], idx_v)
        pltpu.sync_copy(ones_v, hist_sh.at[idx_v], add=True)

    plsc.subcore_barrier()
    r0 = sid * ROWS_PER_S
    pltpu.sync_copy(
        hist_sh.at[pl.ds(r0, ROWS_PER_S)],
        out_hbm.at[cid, pl.ds(r0, ROWS_PER_S)],
    )


def _edge_body(g_hbm, src_hbm, dst_hbm, out_hbm, si_v, di_v, rows_v,
               zbuf_v, acc_sh, sem):
    cid = lax.axis_index("c")
    sid = lax.axis_index("s")
    wid = sid * NC + cid

    @pl.loop(0, ZROWS)
    def _(r):
        for c in range(D // 16):
            zbuf_v[r, pl.ds(c * 16, 16)] = jnp.zeros((16,), jnp.float32)

    @pl.loop(0, ZCOPIES)
    def _(k):
        pltpu.sync_copy(
            zbuf_v, acc_sh.at[pl.ds(sid * ROWS_PER_S + k * ZROWS, ZROWS)])

    plsc.subcore_barrier()

    base = wid * EDGES_PER_W

    @pl.loop(0, NCHUNK)
    def _(j):
        off = pl.multiple_of(base + j * CHUNK, 8)
        pltpu.sync_copy(src_hbm.at[pl.ds(off, CHUNK)], si_v)
        pltpu.sync_copy(dst_hbm.at[pl.ds(off, CHUNK)], di_v)
        pltpu.async_copy(g_hbm.at[si_v], rows_v, sem).wait()
        pltpu.sync_copy(rows_v, acc_sh.at[di_v], add=True)

    plsc.subcore_barrier()

    @pl.loop(0, ZCOPIES)
    def _(k):
        r0 = sid * ROWS_PER_S + k * ZROWS
        pltpu.sync_copy(acc_sh.at[pl.ds(r0, ZROWS)],
                        out_hbm.at[cid, pl.ds(r0, ZROWS)])


@functools.cache
def _sc_kernels():
    mesh = plsc.VectorSubcoreMesh(
        core_axis_name="c", subcore_axis_name="s",
        num_cores=NC, num_subcores=NS)
    deg_kernel = pl.kernel(
        _deg_body,
        out_type=jax.ShapeDtypeStruct((NC, N_PAD, HW), jnp.float32),
        mesh=mesh,
        scratch_types=[
            pltpu.VMEM((CHUNK,), jnp.int32),
            pltpu.VMEM((CHUNK, HW), jnp.float32),
            pltpu.VMEM((ZROWS, HW), jnp.float32),
            pltpu.VMEM_SHARED((N_PAD, HW), jnp.float32),
        ],
    )
    edge_kernel = pl.kernel(
        _edge_body,
        out_type=jax.ShapeDtypeStruct((NC, N_PAD, D), jnp.float32),
        mesh=mesh,
        scratch_types=[
            pltpu.VMEM((CHUNK,), jnp.int32),
            pltpu.VMEM((CHUNK,), jnp.int32),
            pltpu.VMEM((CHUNK, D), jnp.float32),
            pltpu.VMEM((ZROWS, D), jnp.float32),
            pltpu.VMEM_SHARED((N_PAD, D), jnp.float32),
            pltpu.SemaphoreType.DMA,
        ],
    )
    return deg_kernel, edge_kernel


_BLK = 400


def _dis(hist_ref):
    deg = hist_ref[0, :, 0:1] + hist_ref[1, :, 0:1] + 1.0
    return lax.rsqrt(deg)


def _scale_mm_body(hist_ref, x_ref, w_ref, o_ref):
    h = jnp.dot(x_ref[...], w_ref[...], preferred_element_type=jnp.float32)
    o_ref[...] = h * _dis(hist_ref)


def _combine_mm_body(hist_ref, s_ref, g_ref, b_ref, w_ref, o_ref):
    dis = _dis(hist_ref)
    h = dis * (s_ref[0] + s_ref[1] + g_ref[...]) + b_ref[...]
    o_ref[...] = jnp.dot(
        h, w_ref[...], preferred_element_type=jnp.float32) * dis


def _combine_body(hist_ref, s_ref, g_ref, b_ref, o_ref):
    o_ref[...] = _dis(hist_ref) * (s_ref[0] + s_ref[1] + g_ref[...]) \
        + b_ref[...]


_HIST_SPEC = pl.BlockSpec((NC, _BLK, HW), lambda i: (0, i, 0))
_ROW_SPEC = pl.BlockSpec((_BLK, D), lambda i: (i, 0))
_S_SPEC = pl.BlockSpec((NC, _BLK, D), lambda i: (0, i, 0))
_W_SPEC = pl.BlockSpec((D, D), lambda i: (0, 0))
_B_SPEC = pl.BlockSpec((1, D), lambda i: (0, 0))
_OUT = jax.ShapeDtypeStruct((N, D), jnp.float32)


def _scale_mm(hist, x, w):
    return pl.pallas_call(
        _scale_mm_body,
        grid=(N // _BLK,),
        in_specs=[_HIST_SPEC, _ROW_SPEC, _W_SPEC],
        out_specs=_ROW_SPEC,
        out_shape=_OUT,
    )(hist, x, w)


def _combine_mm(hist, s, g, b, w):
    return pl.pallas_call(
        _combine_mm_body,
        grid=(N // _BLK,),
        in_specs=[_HIST_SPEC, _S_SPEC, _ROW_SPEC, _B_SPEC, _W_SPEC],
        out_specs=_ROW_SPEC,
        out_shape=_OUT,
    )(hist, s, g, b, w)


def _combine(hist, s, g, b):
    return pl.pallas_call(
        _combine_body,
        grid=(N // _BLK,),
        in_specs=[_HIST_SPEC, _S_SPEC, _ROW_SPEC, _B_SPEC],
        out_specs=_ROW_SPEC,
        out_shape=_OUT,
    )(hist, s, g, b)


def kernel(x, edge_index, W1, b1, W2, b2):
    src = edge_index[0].astype(jnp.int32)
    dst = edge_index[1].astype(jnp.int32)
    b1r = b1.reshape(1, D)
    b2r = b2.reshape(1, D)

    deg_kernel, edge_kernel = _sc_kernels()
    hist = deg_kernel(dst)
    g1 = _scale_mm(hist, x, W1)
    s1 = edge_kernel(g1, src, dst)
    g2 = _combine_mm(hist, s1, g1, b1r, W2)
    s2 = edge_kernel(g2, src, dst)
    return _combine(hist, s2, g2, b2r)


# speedup vs baseline: 19.4180x; 1.5264x over previous
import functools

import jax
import jax.numpy as jnp
from jax import lax
from jax.experimental import pallas as pl
from jax.experimental.pallas import tpu as pltpu
from jax.experimental.pallas import tpu_sc as plsc

N = 10000
D = 128
E = 320000

NC = 2
NS = 16
NW = NC * NS
EDGES_PER_W = E // NW
CHUNK = 40
NCHUNK = EDGES_PER_W // CHUNK
NBUF = 2
NGROUP = NCHUNK // NBUF
N_PAD = 10240
ROWS_PER_S = N_PAD // NS
ZROWS = 128
ZCOPIES = ROWS_PER_S // ZROWS
EZROWS = 16
EZCOPIES = ROWS_PER_S // EZROWS
HW = 16
DBLK = 25
DNBLK = NCHUNK // DBLK


def _deg_body(dstb_hbm, out_hbm, idx_v, ones_v, zbuf_v, hist_sh):
    cid = lax.axis_index("c")
    sid = lax.axis_index("s")
    wid = sid * NC + cid

    @pl.loop(0, CHUNK)
    def _(r):
        ones_v[r, :] = jnp.ones((HW,), jnp.float32)

    @pl.loop(0, ZROWS)
    def _(r):
        zbuf_v[r, :] = jnp.zeros((HW,), jnp.float32)

    @pl.loop(0, ZCOPIES)
    def _(k):
        pltpu.sync_copy(
            zbuf_v, hist_sh.at[pl.ds(sid * ROWS_PER_S + k * ZROWS, ZROWS)])

    plsc.subcore_barrier()

    @pl.loop(0, DNBLK)
    def _(bk):
        pltpu.sync_copy(dstb_hbm.at[wid, bk], idx_v)

        @pl.loop(0, DBLK)
        def _(j):
            pltpu.sync_copy(ones_v, hist_sh.at[idx_v.at[j]], add=True)

    plsc.subcore_barrier()
    r0 = sid * ROWS_PER_S
    pltpu.sync_copy(
        hist_sh.at[pl.ds(r0, ROWS_PER_S)],
        out_hbm.at[cid, pl.ds(r0, ROWS_PER_S)],
    )


def _edge_body(g_hbm, gidx_hbm, out_hbm, idx_v, rows_v, zbuf_v, acc_sh,
               isem, gsem, ssem):
    cid = lax.axis_index("c")
    sid = lax.axis_index("s")
    wid = sid * NC + cid

    @pl.loop(0, EZROWS)
    def _(r):
        for c in range(D // 16):
            zbuf_v[r, pl.ds(c * 16, 16)] = jnp.zeros((16,), jnp.float32)

    @pl.loop(0, EZCOPIES)
    def _(k):
        pltpu.sync_copy(
            zbuf_v, acc_sh.at[pl.ds(sid * ROWS_PER_S + k * EZROWS, EZROWS)])

    plsc.subcore_barrier()

    def start_idx(g, p):
        pltpu.async_copy(gidx_hbm.at[wid, g], idx_v.at[p], isem.at[p])

    def wait_idx(g, p):
        pltpu.make_async_copy(
            gidx_hbm.at[wid, g], idx_v.at[p], isem.at[p]).wait()

    def start_gather(p, b):
        pltpu.async_copy(
            g_hbm.at[idx_v.at[p, 0, b]], rows_v.at[b], gsem.at[b])

    def wait_gather(p, b):
        pltpu.make_async_copy(
            g_hbm.at[idx_v.at[p, 0, b]], rows_v.at[b], gsem.at[b]).wait()

    def start_scatter(p, b):
        pltpu.async_copy(
            rows_v.at[b], acc_sh.at[idx_v.at[p, 1, b]], ssem.at[b],
            add=True)

    def wait_scatter(p, b):
        pltpu.make_async_copy(
            rows_v.at[b], acc_sh.at[idx_v.at[p, 1, b]], ssem.at[b]).wait()

    start_idx(0, 0)
    wait_idx(0, 0)
    for b in range(NBUF):
        start_gather(0, b)
    start_idx(1, 1)

    @pl.loop(0, NGROUP)
    def _(g):
        p = lax.rem(g, 2)
        for b in range(NBUF):
            wait_gather(p, b)
            start_scatter(p, b)
        for b in range(NBUF):
            wait_scatter(p, b)

        @pl.when(g + 1 < NGROUP)
        def _():
            p1 = 1 - p
            wait_idx(g + 1, p1)
            for b in range(NBUF):
                start_gather(p1, b)

            @pl.when(g + 2 < NGROUP)
            def _():
                start_idx(g + 2, p)

    plsc.subcore_barrier()

    @pl.loop(0, ZCOPIES)
    def _(k):
        r0 = sid * ROWS_PER_S + k * ZROWS
        pltpu.sync_copy(acc_sh.at[pl.ds(r0, ZROWS)],
                        out_hbm.at[cid, pl.ds(r0, ZROWS)])


@functools.cache
def _sc_kernels():
    mesh = plsc.VectorSubcoreMesh(
        core_axis_name="c", subcore_axis_name="s",
        num_cores=NC, num_subcores=NS)
    deg_kernel = pl.kernel(
        _deg_body,
        out_type=jax.ShapeDtypeStruct((NC, N_PAD, HW), jnp.float32),
        mesh=mesh,
        scratch_types=[
            pltpu.VMEM((DBLK, CHUNK), jnp.int32),
            pltpu.VMEM((CHUNK, HW), jnp.float32),
            pltpu.VMEM((ZROWS, HW), jnp.float32),
            pltpu.VMEM_SHARED((N_PAD, HW), jnp.float32),
        ],
    )
    edge_kernel = pl.kernel(
        _edge_body,
        out_type=jax.ShapeDtypeStruct((NC, N_PAD, D), jnp.float32),
        mesh=mesh,
        scratch_types=[
            pltpu.VMEM((2, 2, NBUF, CHUNK), jnp.int32),
            pltpu.VMEM((NBUF, CHUNK, D), jnp.float32),
            pltpu.VMEM((EZROWS, D), jnp.float32),
            pltpu.VMEM_SHARED((N_PAD, D), jnp.float32),
            pltpu.SemaphoreType.DMA((2,)),
            pltpu.SemaphoreType.DMA((NBUF,)),
            pltpu.SemaphoreType.DMA((NBUF,)),
        ],
    )
    return deg_kernel, edge_kernel


_BLK = 400


def _dis(hist_ref):
    deg = hist_ref[0, :, 0:1] + hist_ref[1, :, 0:1] + 1.0
    return lax.rsqrt(deg)


def _scale_mm_body(hist_ref, x_ref, w_ref, o_ref):
    h = jnp.dot(x_ref[...], w_ref[...], preferred_element_type=jnp.float32)
    o_ref[...] = h * _dis(hist_ref)


def _combine_mm_body(hist_ref, s_ref, g_ref, b_ref, w_ref, o_ref):
    dis = _dis(hist_ref)
    h = dis * (s_ref[0] + s_ref[1] + g_ref[...]) + b_ref[...]
    o_ref[...] = jnp.dot(
        h, w_ref[...], preferred_element_type=jnp.float32) * dis


def _combine_body(hist_ref, s_ref, g_ref, b_ref, o_ref):
    o_ref[...] = _dis(hist_ref) * (s_ref[0] + s_ref[1] + g_ref[...]) \
        + b_ref[...]


_HIST_SPEC = pl.BlockSpec((NC, _BLK, HW), lambda i: (0, i, 0))
_ROW_SPEC = pl.BlockSpec((_BLK, D), lambda i: (i, 0))
_S_SPEC = pl.BlockSpec((NC, _BLK, D), lambda i: (0, i, 0))
_W_SPEC = pl.BlockSpec((D, D), lambda i: (0, 0))
_B_SPEC = pl.BlockSpec((1, D), lambda i: (0, 0))
_OUT = jax.ShapeDtypeStruct((N, D), jnp.float32)


def _scale_mm(hist, x, w):
    return pl.pallas_call(
        _scale_mm_body,
        grid=(N // _BLK,),
        in_specs=[_HIST_SPEC, _ROW_SPEC, _W_SPEC],
        out_specs=_ROW_SPEC,
        out_shape=_OUT,
    )(hist, x, w)


def _combine_mm(hist, s, g, b, w):
    return pl.pallas_call(
        _combine_mm_body,
        grid=(N // _BLK,),
        in_specs=[_HIST_SPEC, _S_SPEC, _ROW_SPEC, _B_SPEC, _W_SPEC],
        out_specs=_ROW_SPEC,
        out_shape=_OUT,
    )(hist, s, g, b, w)


def _combine(hist, s, g, b):
    return pl.pallas_call(
        _combine_body,
        grid=(N // _BLK,),
        in_specs=[_HIST_SPEC, _S_SPEC, _ROW_SPEC, _B_SPEC],
        out_specs=_ROW_SPEC,
        out_shape=_OUT,
    )(hist, s, g, b)


def kernel(x, edge_index, W1, b1, W2, b2):
    eidx = edge_index.astype(jnp.int32)
    gidx = jnp.transpose(
        eidx.reshape(2, NW, NGROUP, NBUF, CHUNK), (1, 2, 0, 3, 4))
    dstb = eidx[1].reshape(NW, DNBLK, DBLK, CHUNK)
    b1r = b1.reshape(1, D)
    b2r = b2.reshape(1, D)

    deg_kernel, edge_kernel = _sc_kernels()
    hist = deg_kernel(dstb)
    g1 = _scale_mm(hist, x, W1)
    s1 = edge_kernel(g1, gidx)
    g2 = _combine_mm(hist, s1, g1, b1r, W2)
    s2 = edge_kernel(g2, gidx)
    return _combine(hist, s2, g2, b2r)


# speedup vs baseline: 23.2634x; 1.1980x over previous
import functools

import jax
import jax.numpy as jnp
from jax import lax
from jax.experimental import pallas as pl
from jax.experimental.pallas import tpu as pltpu
from jax.experimental.pallas import tpu_sc as plsc

N = 10000
D = 128
E = 320000

NC = 2
NS = 16
NW = NC * NS
EDGES_PER_W = E // NW
CHUNK = 40
NCHUNK = EDGES_PER_W // CHUNK
NBUF = 5
NGROUP = NCHUNK // NBUF
N_PAD = 10240
ROWS_PER_S = N_PAD // NS
ZROWS = 128
ZCOPIES = ROWS_PER_S // ZROWS
EZROWS = 16
EZCOPIES = ROWS_PER_S // EZROWS
HW = 16
DBLK = 25
DNBLK = NCHUNK // DBLK


def _deg_body(dstb_hbm, out_hbm, idx_v, ones_v, zbuf_v, hist_sh):
    cid = lax.axis_index("c")
    sid = lax.axis_index("s")
    wid = sid * NC + cid

    @pl.loop(0, CHUNK)
    def _(r):
        ones_v[r, :] = jnp.ones((HW,), jnp.float32)

    @pl.loop(0, ZROWS)
    def _(r):
        zbuf_v[r, :] = jnp.zeros((HW,), jnp.float32)

    @pl.loop(0, ZCOPIES)
    def _(k):
        pltpu.sync_copy(
            zbuf_v, hist_sh.at[pl.ds(sid * ROWS_PER_S + k * ZROWS, ZROWS)])

    plsc.subcore_barrier()

    @pl.loop(0, DNBLK)
    def _(bk):
        pltpu.sync_copy(dstb_hbm.at[wid, bk], idx_v)

        @pl.loop(0, DBLK)
        def _(j):
            pltpu.sync_copy(ones_v, hist_sh.at[idx_v.at[j]], add=True)

    plsc.subcore_barrier()
    r0 = sid * ROWS_PER_S
    pltpu.sync_copy(
        hist_sh.at[pl.ds(r0, ROWS_PER_S)],
        out_hbm.at[cid, pl.ds(r0, ROWS_PER_S)],
    )


def _edge_body(g_hbm, gidx_hbm, out_hbm, idx_v, rows_v, zbuf_v, acc_sh,
               isem, gsem, ssem):
    cid = lax.axis_index("c")
    sid = lax.axis_index("s")
    wid = sid * NC + cid

    @pl.loop(0, EZROWS)
    def _(r):
        for c in range(D // 16):
            zbuf_v[r, pl.ds(c * 16, 16)] = jnp.zeros((16,), jnp.float32)

    @pl.loop(0, EZCOPIES)
    def _(k):
        pltpu.sync_copy(
            zbuf_v, acc_sh.at[pl.ds(sid * ROWS_PER_S + k * EZROWS, EZROWS)])

    plsc.subcore_barrier()

    def start_idx(g, p):
        pltpu.async_copy(gidx_hbm.at[wid, g], idx_v.at[p], isem.at[p])

    def wait_idx(g, p):
        pltpu.make_async_copy(
            gidx_hbm.at[wid, g], idx_v.at[p], isem.at[p]).wait()

    def start_gather(p, b):
        pltpu.async_copy(
            g_hbm.at[idx_v.at[p, 0, b]], rows_v.at[b], gsem.at[b])

    def wait_gather(p, b):
        pltpu.make_async_copy(
            g_hbm.at[idx_v.at[p, 0, b]], rows_v.at[b], gsem.at[b]).wait()

    def start_scatter(p, b):
        pltpu.async_copy(
            rows_v.at[b], acc_sh.at[idx_v.at[p, 1, b]], ssem.at[b],
            add=True)

    def wait_scatter(p, b):
        pltpu.make_async_copy(
            rows_v.at[b], acc_sh.at[idx_v.at[p, 1, b]], ssem.at[b]).wait()

    start_idx(0, 0)
    wait_idx(0, 0)
    for b in range(NBUF):
        start_gather(0, b)
    start_idx(1, 1)

    @pl.loop(0, NGROUP)
    def _(g):
        p = lax.rem(g, 2)
        for b in range(NBUF):
            wait_gather(p, b)
            start_scatter(p, b)
        for b in range(NBUF):
            wait_scatter(p, b)

        @pl.when(g + 1 < NGROUP)
        def _():
            p1 = 1 - p
            wait_idx(g + 1, p1)
            for b in range(NBUF):
                start_gather(p1, b)

            @pl.when(g + 2 < NGROUP)
            def _():
                start_idx(g + 2, p)

    plsc.subcore_barrier()

    @pl.loop(0, ZCOPIES)
    def _(k):
        r0 = sid * ROWS_PER_S + k * ZROWS
        pltpu.sync_copy(acc_sh.at[pl.ds(r0, ZROWS)],
                        out_hbm.at[cid, pl.ds(r0, ZROWS)])


@functools.cache
def _sc_kernels():
    mesh = plsc.VectorSubcoreMesh(
        core_axis_name="c", subcore_axis_name="s",
        num_cores=NC, num_subcores=NS)
    deg_kernel = pl.kernel(
        _deg_body,
        out_type=jax.ShapeDtypeStruct((NC, N_PAD, HW), jnp.float32),
        mesh=mesh,
        scratch_types=[
            pltpu.VMEM((DBLK, CHUNK), jnp.int32),
            pltpu.VMEM((CHUNK, HW), jnp.float32),
            pltpu.VMEM((ZROWS, HW), jnp.float32),
            pltpu.VMEM_SHARED((N_PAD, HW), jnp.float32),
        ],
    )
    edge_kernel = pl.kernel(
        _edge_body,
        out_type=jax.ShapeDtypeStruct((NC, N_PAD, D), jnp.float32),
        mesh=mesh,
        scratch_types=[
            pltpu.VMEM((2, 2, NBUF, CHUNK), jnp.int32),
            pltpu.VMEM((NBUF, CHUNK, D), jnp.float32),
            pltpu.VMEM((EZROWS, D), jnp.float32),
            pltpu.VMEM_SHARED((N_PAD, D), jnp.float32),
            pltpu.SemaphoreType.DMA((2,)),
            pltpu.SemaphoreType.DMA((NBUF,)),
            pltpu.SemaphoreType.DMA((NBUF,)),
        ],
    )
    return deg_kernel, edge_kernel


_BLK = 400


def _dis(hist_ref):
    deg = hist_ref[0, :, 0:1] + hist_ref[1, :, 0:1] + 1.0
    return lax.rsqrt(deg)


def _scale_mm_body(hist_ref, x_ref, w_ref, o_ref):
    h = jnp.dot(x_ref[...], w_ref[...], preferred_element_type=jnp.float32)
    o_ref[...] = h * _dis(hist_ref)


def _combine_mm_body(hist_ref, s_ref, g_ref, b_ref, w_ref, o_ref):
    dis = _dis(hist_ref)
    h = dis * (s_ref[0] + s_ref[1] + g_ref[...]) + b_ref[...]
    o_ref[...] = jnp.dot(
        h, w_ref[...], preferred_element_type=jnp.float32) * dis


def _combine_body(hist_ref, s_ref, g_ref, b_ref, o_ref):
    o_ref[...] = _dis(hist_ref) * (s_ref[0] + s_ref[1] + g_ref[...]) \
        + b_ref[...]


_HIST_SPEC = pl.BlockSpec((NC, _BLK, HW), lambda i: (0, i, 0))
_ROW_SPEC = pl.BlockSpec((_BLK, D), lambda i: (i, 0))
_S_SPEC = pl.BlockSpec((NC, _BLK, D), lambda i: (0, i, 0))
_W_SPEC = pl.BlockSpec((D, D), lambda i: (0, 0))
_B_SPEC = pl.BlockSpec((1, D), lambda i: (0, 0))
_OUT = jax.ShapeDtypeStruct((N, D), jnp.float32)


def _scale_mm(hist, x, w):
    return pl.pallas_call(
        _scale_mm_body,
        grid=(N // _BLK,),
        in_specs=[_HIST_SPEC, _ROW_SPEC, _W_SPEC],
        out_specs=_ROW_SPEC,
        out_shape=_OUT,
    )(hist, x, w)


def _combine_mm(hist, s, g, b, w):
    return pl.pallas_call(
        _combine_mm_body,
        grid=(N // _BLK,),
        in_specs=[_HIST_SPEC, _S_SPEC, _ROW_SPEC, _B_SPEC, _W_SPEC],
        out_specs=_ROW_SPEC,
        out_shape=_OUT,
    )(hist, s, g, b, w)


def _combine(hist, s, g, b):
    return pl.pallas_call(
        _combine_body,
        grid=(N // _BLK,),
        in_specs=[_HIST_SPEC, _S_SPEC, _ROW_SPEC, _B_SPEC],
        out_specs=_ROW_SPEC,
        out_shape=_OUT,
    )(hist, s, g, b)


def kernel(x, edge_index, W1, b1, W2, b2):
    eidx = edge_index.astype(jnp.int32)
    gidx = jnp.transpose(
        eidx.reshape(2, NW, NGROUP, NBUF, CHUNK), (1, 2, 0, 3, 4))
    dstb = eidx[1].reshape(NW, DNBLK, DBLK, CHUNK)
    b1r = b1.reshape(1, D)
    b2r = b2.reshape(1, D)

    deg_kernel, edge_kernel = _sc_kernels()
    hist = deg_kernel(dstb)
    g1 = _scale_mm(hist, x, W1)
    s1 = edge_kernel(g1, gidx)
    g2 = _combine_mm(hist, s1, g1, b1r, W2)
    s2 = edge_kernel(g2, gidx)
    return _combine(hist, s2, g2, b2r)


# speedup vs baseline: 28.2278x; 1.2134x over previous
import functools

import jax
import jax.numpy as jnp
from jax import lax
from jax.experimental import pallas as pl
from jax.experimental.pallas import tpu as pltpu
from jax.experimental.pallas import tpu_sc as plsc

N = 10000
D = 128
E = 320000

NC = 2
NS = 16
NW = NC * NS
EDGES_PER_W = E // NW
CHUNK = 40
NCHUNK = EDGES_PER_W // CHUNK
NBUF = 5
NGROUP = NCHUNK // NBUF
N_PAD = 10240
ROWS_PER_S = N_PAD // NS
ZROWS = 128
ZCOPIES = ROWS_PER_S // ZROWS
EZROWS = 16
EZCOPIES = ROWS_PER_S // EZROWS
HW = 16
DBLK = 25
DNBLK = NCHUNK // DBLK


def _deg_body(dstb_hbm, out_hbm, idx_v, ones_v, zbuf_v, hist_sh, dsem):
    cid = lax.axis_index("c")
    sid = lax.axis_index("s")
    wid = sid * NC + cid

    @pl.loop(0, CHUNK)
    def _(r):
        ones_v[r, :] = jnp.ones((HW,), jnp.float32)

    @pl.loop(0, ZROWS)
    def _(r):
        zbuf_v[r, :] = jnp.zeros((HW,), jnp.float32)

    @pl.loop(0, ZCOPIES)
    def _(k):
        pltpu.sync_copy(
            zbuf_v, hist_sh.at[pl.ds(sid * ROWS_PER_S + k * ZROWS, ZROWS)])

    plsc.subcore_barrier()

    @pl.loop(0, DNBLK)
    def _(bk):
        pltpu.sync_copy(dstb_hbm.at[wid, bk], idx_v)

        @pl.loop(0, DBLK)
        def _(j):
            pltpu.async_copy(ones_v, hist_sh.at[idx_v.at[j]], dsem,
                             add=True)

        @pl.loop(0, DBLK)
        def _(j):
            pltpu.make_async_copy(
                ones_v, hist_sh.at[idx_v.at[j]], dsem).wait()

    plsc.subcore_barrier()
    r0 = sid * ROWS_PER_S
    pltpu.sync_copy(
        hist_sh.at[pl.ds(r0, ROWS_PER_S)],
        out_hbm.at[cid, pl.ds(r0, ROWS_PER_S)],
    )


def _edge_body(g_hbm, gidx_hbm, out_hbm, idx_v, rows_v, zbuf_v, acc_sh,
               isem, gsem, ssem):
    cid = lax.axis_index("c")
    sid = lax.axis_index("s")
    wid = sid * NC + cid

    @pl.loop(0, EZROWS)
    def _(r):
        for c in range(D // 16):
            zbuf_v[r, pl.ds(c * 16, 16)] = jnp.zeros((16,), jnp.float32)

    @pl.loop(0, EZCOPIES)
    def _(k):
        pltpu.sync_copy(
            zbuf_v, acc_sh.at[pl.ds(sid * ROWS_PER_S + k * EZROWS, EZROWS)])

    plsc.subcore_barrier()

    def start_idx(g, p):
        pltpu.async_copy(gidx_hbm.at[wid, g], idx_v.at[p], isem.at[p])

    def wait_idx(g, p):
        pltpu.make_async_copy(
            gidx_hbm.at[wid, g], idx_v.at[p], isem.at[p]).wait()

    def start_gather(p, b):
        pltpu.async_copy(
            g_hbm.at[idx_v.at[p, 0, b]], rows_v.at[b], gsem.at[b])

    def wait_gather(p, b):
        pltpu.make_async_copy(
            g_hbm.at[idx_v.at[p, 0, b]], rows_v.at[b], gsem.at[b]).wait()

    def start_scatter(p, b):
        pltpu.async_copy(
            rows_v.at[b], acc_sh.at[idx_v.at[p, 1, b]], ssem.at[b],
            add=True)

    def wait_scatter(p, b):
        pltpu.make_async_copy(
            rows_v.at[b], acc_sh.at[idx_v.at[p, 1, b]], ssem.at[b]).wait()

    start_idx(0, 0)
    wait_idx(0, 0)
    for b in range(NBUF):
        start_gather(0, b)
    start_idx(1, 1)

    @pl.loop(0, NGROUP)
    def _(g):
        p = lax.rem(g, 2)
        p1 = 1 - p
        for b in range(NBUF):
            wait_gather(p, b)
            start_scatter(p, b)

        @pl.when(g + 1 < NGROUP)
        def _():
            wait_idx(g + 1, p1)
            for b in range(NBUF):
                wait_scatter(p, b)
                start_gather(p1, b)

            @pl.when(g + 2 < NGROUP)
            def _():
                start_idx(g + 2, p)

        @pl.when(g + 1 >= NGROUP)
        def _():
            for b in range(NBUF):
                wait_scatter(p, b)

    plsc.subcore_barrier()

    @pl.loop(0, ZCOPIES)
    def _(k):
        r0 = sid * ROWS_PER_S + k * ZROWS
        pltpu.sync_copy(acc_sh.at[pl.ds(r0, ZROWS)],
                        out_hbm.at[cid, pl.ds(r0, ZROWS)])


@functools.cache
def _sc_kernels():
    mesh = plsc.VectorSubcoreMesh(
        core_axis_name="c", subcore_axis_name="s",
        num_cores=NC, num_subcores=NS)
    deg_kernel = pl.kernel(
        _deg_body,
        out_type=jax.ShapeDtypeStruct((NC, N_PAD, HW), jnp.float32),
        mesh=mesh,
        scratch_types=[
            pltpu.VMEM((DBLK, CHUNK), jnp.int32),
            pltpu.VMEM((CHUNK, HW), jnp.float32),
            pltpu.VMEM((ZROWS, HW), jnp.float32),
            pltpu.VMEM_SHARED((N_PAD, HW), jnp.float32),
            pltpu.SemaphoreType.DMA,
        ],
    )
    edge_kernel = pl.kernel(
        _edge_body,
        out_type=jax.ShapeDtypeStruct((NC, N_PAD, D), jnp.float32),
        mesh=mesh,
        scratch_types=[
            pltpu.VMEM((2, 2, NBUF, CHUNK), jnp.int32),
            pltpu.VMEM((NBUF, CHUNK, D), jnp.float32),
            pltpu.VMEM((EZROWS, D), jnp.float32),
            pltpu.VMEM_SHARED((N_PAD, D), jnp.float32),
            pltpu.SemaphoreType.DMA((2,)),
            pltpu.SemaphoreType.DMA((NBUF,)),
            pltpu.SemaphoreType.DMA((NBUF,)),
        ],
    )
    return deg_kernel, edge_kernel


_BLK = 400


def _dis(hist_ref):
    deg = hist_ref[0, :, 0:1] + hist_ref[1, :, 0:1] + 1.0
    return lax.rsqrt(deg)


def _scale_mm_body(hist_ref, x_ref, w_ref, o_ref):
    h = jnp.dot(x_ref[...], w_ref[...], preferred_element_type=jnp.float32)
    o_ref[...] = h * _dis(hist_ref)


def _combine_mm_body(hist_ref, s_ref, g_ref, b_ref, w_ref, o_ref):
    dis = _dis(hist_ref)
    h = dis * (s_ref[0] + s_ref[1] + g_ref[...]) + b_ref[...]
    o_ref[...] = jnp.dot(
        h, w_ref[...], preferred_element_type=jnp.float32) * dis


def _combine_body(hist_ref, s_ref, g_ref, b_ref, o_ref):
    o_ref[...] = _dis(hist_ref) * (s_ref[0] + s_ref[1] + g_ref[...]) \
        + b_ref[...]


_HIST_SPEC = pl.BlockSpec((NC, _BLK, HW), lambda i: (0, i, 0))
_ROW_SPEC = pl.BlockSpec((_BLK, D), lambda i: (i, 0))
_S_SPEC = pl.BlockSpec((NC, _BLK, D), lambda i: (0, i, 0))
_W_SPEC = pl.BlockSpec((D, D), lambda i: (0, 0))
_B_SPEC = pl.BlockSpec((1, D), lambda i: (0, 0))
_OUT = jax.ShapeDtypeStruct((N, D), jnp.float32)


def _scale_mm(hist, x, w):
    return pl.pallas_call(
        _scale_mm_body,
        grid=(N // _BLK,),
        in_specs=[_HIST_SPEC, _ROW_SPEC, _W_SPEC],
        out_specs=_ROW_SPEC,
        out_shape=_OUT,
    )(hist, x, w)


def _combine_mm(hist, s, g, b, w):
    return pl.pallas_call(
        _combine_mm_body,
        grid=(N // _BLK,),
        in_specs=[_HIST_SPEC, _S_SPEC, _ROW_SPEC, _B_SPEC, _W_SPEC],
        out_specs=_ROW_SPEC,
        out_shape=_OUT,
    )(hist, s, g, b, w)


def _combine(hist, s, g, b):
    return pl.pallas_call(
        _combine_body,
        grid=(N // _BLK,),
        in_specs=[_HIST_SPEC, _S_SPEC, _ROW_SPEC, _B_SPEC],
        out_specs=_ROW_SPEC,
        out_shape=_OUT,
    )(hist, s, g, b)


def kernel(x, edge_index, W1, b1, W2, b2):
    eidx = edge_index.astype(jnp.int32)
    gidx = jnp.transpose(
        eidx.reshape(2, NW, NGROUP, NBUF, CHUNK), (1, 2, 0, 3, 4))
    dstb = eidx[1].reshape(NW, DNBLK, DBLK, CHUNK)
    b1r = b1.reshape(1, D)
    b2r = b2.reshape(1, D)

    deg_kernel, edge_kernel = _sc_kernels()
    hist = deg_kernel(dstb)
    g1 = _scale_mm(hist, x, W1)
    s1 = edge_kernel(g1, gidx)
    g2 = _combine_mm(hist, s1, g1, b1r, W2)
    s2 = edge_kernel(g2, gidx)
    return _combine(hist, s2, g2, b2r)
